# baseline (speedup 1.0000x reference)
import functools

import jax
import jax.numpy as jnp
from jax import lax
from jax.experimental import pallas as pl
from jax.experimental.pallas import tpu as pltpu
from jax.experimental.pallas import tpu_sc as plsc

VOCAB_SIZE = 1000000
EMB_DIM = 64
SUBLANES = 8
LANES = 16
NUM_CORES = 2
NUM_SUBCORES = 16
NUM_WORKERS = NUM_CORES * NUM_SUBCORES


def _sc_embed_body(words_hbm, table_hbm, out_hbm, idx_v, tile_v, row_v, oov_v,
                   rows_v, sem):
    b_per_w = idx_v.shape[0]
    wid = lax.axis_index("s") * NUM_CORES + lax.axis_index("c")
    base = wid * b_per_w

    pltpu.sync_copy(words_hbm.at[pl.ds(base, b_per_w)], idx_v)

    n_groups = b_per_w // LANES
    lane_iota = lax.iota(jnp.int32, LANES)
    zeros_i = jnp.zeros((LANES,), jnp.int32)
    ones_i = jnp.ones((LANES,), jnp.int32)

    def group_step(g, n_oov):
        v = idx_v[pl.ds(g * LANES, LANES)]
        in_vocab = v < VOCAB_SIZE
        safe = jnp.where(in_vocab, v, 0)
        tile_v[pl.ds(g * LANES, LANES)] = lax.shift_right_logical(safe, 3)
        row_v[pl.ds(g * LANES, LANES)] = lax.bitwise_and(safe, 7)
        oov = v >= VOCAB_SIZE
        oov_i32 = jnp.where(in_vocab, zeros_i, ones_i)
        pos = plsc.cumsum(oov_i32)
        tgt = jnp.full((LANES,), n_oov, jnp.int32) + pos - ones_i
        rowids = lane_iota + jnp.full((LANES,), g * LANES, jnp.int32)
        plsc.store_scatter(oov_v, [tgt], rowids, mask=oov)
        return n_oov + jnp.sum(oov_i32)

    n_oov = lax.fori_loop(0, n_groups, group_step, jnp.int32(0))

    def fetch(g, carry):
        t16 = tile_v[pl.ds(g * LANES, LANES)]
        r16 = row_v[pl.ds(g * LANES, LANES)]
        for k in range(LANES):
            pltpu.async_copy(table_hbm.at[t16[k], r16[k]],
                             rows_v.at[g * LANES + k], sem)
        return carry

    lax.fori_loop(0, n_groups, fetch, jnp.int32(0))

    def drain(j, carry):
        pltpu.make_async_copy(table_hbm.at[0, 0], rows_v.at[0], sem).wait()
        return carry

    lax.fori_loop(0, b_per_w, drain, jnp.int32(0))

    zeros_f = jnp.zeros((LANES,), jnp.float32)

    def zero_chunk(c, carry):
        r = oov_v[pl.ds(c * LANES, LANES)]
        lane_pos = lane_iota + jnp.full((LANES,), c * LANES, jnp.int32)
        valid = lane_pos < jnp.full((LANES,), n_oov, jnp.int32)
        for d in range(EMB_DIM):
            plsc.store_scatter(rows_v, [r, jnp.full((LANES,), d, jnp.int32)],
                               zeros_f, mask=valid)
        return carry

    lax.fori_loop(0, (n_oov + LANES - 1) // LANES, zero_chunk, jnp.int32(0))

    pltpu.sync_copy(rows_v, out_hbm.at[pl.ds(base, b_per_w)])


def kernel(words, table):
    batch = words.shape[0]
    b_per_w = batch // NUM_WORKERS
    vocab = table.shape[0]
    table3 = table.reshape(vocab // SUBLANES, SUBLANES, EMB_DIM)
    mesh = plsc.VectorSubcoreMesh(core_axis_name="c", subcore_axis_name="s")
    run = pl.kernel(
        _sc_embed_body,
        out_type=jax.ShapeDtypeStruct((batch, EMB_DIM), jnp.float32),
        mesh=mesh,
        scratch_types=[
            pltpu.VMEM((b_per_w,), jnp.int32),
            pltpu.VMEM((b_per_w,), jnp.int32),
            pltpu.VMEM((b_per_w,), jnp.int32),
            pltpu.VMEM((b_per_w + LANES,), jnp.int32),
            pltpu.VMEM((b_per_w, EMB_DIM), jnp.float32),
            pltpu.SemaphoreType.DMA,
        ],
        compiler_params=pltpu.CompilerParams(use_tc_tiling_on_sc=False,
                                             needs_layout_passes=False),
    )
    return run(words, table3)


# speedup vs baseline: 2.2910x; 2.2910x over previous
import functools

import jax
import jax.numpy as jnp
from jax import lax
from jax.experimental import pallas as pl
from jax.experimental.pallas import tpu as pltpu
from jax.experimental.pallas import tpu_sc as plsc

VOCAB_SIZE = 1000000
EMB_DIM = 64
SUBLANES = 8
LANES = 16
NUM_CORES = 2
NUM_SUBCORES = 16
NUM_WORKERS = NUM_CORES * NUM_SUBCORES


def _sc_embed_body(words_hbm, table_hbm, out_hbm, idx_v, tile_v, row_v, oov_v,
                   rows_v, sem):
    b_per_w = idx_v.shape[0]
    wid = lax.axis_index("s") * NUM_CORES + lax.axis_index("c")
    base = wid * b_per_w

    pltpu.sync_copy(words_hbm.at[pl.ds(base, b_per_w)], idx_v)

    n_groups = b_per_w // LANES
    lane_iota = lax.iota(jnp.int32, LANES)
    zeros_i = jnp.zeros((LANES,), jnp.int32)
    ones_i = jnp.ones((LANES,), jnp.int32)

    def group_step(g, n_oov):
        v = idx_v[pl.ds(g * LANES, LANES)]
        in_vocab = v < VOCAB_SIZE
        safe = jnp.where(in_vocab, v, 0)
        tile_v[pl.ds(g * LANES, LANES)] = lax.shift_right_logical(safe, 3)
        row_v[pl.ds(g * LANES, LANES)] = lax.bitwise_and(safe, 7)
        oov = v >= VOCAB_SIZE
        oov_i32 = jnp.where(in_vocab, zeros_i, ones_i)
        pos = plsc.cumsum(oov_i32)
        tgt = jnp.full((LANES,), n_oov, jnp.int32) + pos - ones_i
        rowids = lane_iota + jnp.full((LANES,), g * LANES, jnp.int32)
        plsc.store_scatter(oov_v, [tgt], rowids, mask=oov)
        return n_oov + jnp.sum(oov_i32)

    n_oov = lax.fori_loop(0, n_groups, group_step, jnp.int32(0))

    def fetch(g, carry):
        t16 = tile_v[pl.ds(g * LANES, LANES)]
        r16 = row_v[pl.ds(g * LANES, LANES)]
        for k in range(LANES):
            pltpu.async_copy(table_hbm.at[t16[k], r16[k]],
                             rows_v.at[g * LANES + k], sem)
        return carry

    lax.fori_loop(0, n_groups, fetch, jnp.int32(0))

    def drain(j, carry):
        pltpu.make_async_copy(table_hbm.at[0, 0], rows_v.at[0], sem).wait()
        return carry

    lax.fori_loop(0, b_per_w, drain, jnp.int32(0))

    zeros_f = jnp.zeros((LANES,), jnp.float32)

    def zero_chunk(c, carry):
        r = oov_v[pl.ds(c * LANES, LANES)]
        lane_pos = lane_iota + jnp.full((LANES,), c * LANES, jnp.int32)
        valid = lane_pos < jnp.full((LANES,), n_oov, jnp.int32)
        for d in range(EMB_DIM):
            plsc.store_scatter(rows_v, [r, jnp.full((LANES,), d, jnp.int32)],
                               zeros_f, mask=valid)
        return carry

    lax.fori_loop(0, (n_oov + LANES - 1) // LANES, zero_chunk, jnp.int32(0))

    pltpu.sync_copy(rows_v, out_hbm.at[pl.ds(base, b_per_w)])


def kernel(words, table):
    batch = words.shape[0]
    b_per_w = batch // NUM_WORKERS
    vocab = table.shape[0]
    table3 = table.reshape(vocab // SUBLANES, SUBLANES, EMB_DIM)
    mesh = plsc.VectorSubcoreMesh(core_axis_name="c", subcore_axis_name="s")
    run = pl.kernel(
        _sc_embed_body,
        out_type=jax.ShapeDtypeStruct((batch, EMB_DIM), jnp.float32),
        mesh=mesh,
        scratch_types=[
            pltpu.VMEM((b_per_w,), jnp.int32),
            pltpu.VMEM((b_per_w,), jnp.int32),
            pltpu.VMEM((b_per_w,), jnp.int32),
            pltpu.VMEM((b_per_w + LANES,), jnp.int32),
            pltpu.VMEM((b_per_w, EMB_DIM), jnp.float32),
            pltpu.SemaphoreType.DMA,
        ],
        compiler_params=pltpu.CompilerParams(needs_layout_passes=False),
    )
    return run(words, table3)


# speedup vs baseline: 2.3068x; 1.0069x over previous
import functools

import jax
import jax.numpy as jnp
from jax import lax
from jax.experimental import pallas as pl
from jax.experimental.pallas import tpu as pltpu
from jax.experimental.pallas import tpu_sc as plsc

VOCAB_SIZE = 1000000
EMB_DIM = 64
SUBLANES = 8
LANES = 16
NUM_CORES = 2
NUM_SUBCORES = 16
NUM_WORKERS = NUM_CORES * NUM_SUBCORES


def _sc_embed_body(words_hbm, table_hbm, out_hbm, idx_v, tile_v, row_v, oov_v,
                   rows_v, sem):
    b_per_w = idx_v.shape[0]
    wid = lax.axis_index("s") * NUM_CORES + lax.axis_index("c")
    base = wid * b_per_w

    pltpu.sync_copy(words_hbm.at[pl.ds(base, b_per_w)], idx_v)

    n_groups = b_per_w // LANES
    lane_iota = lax.iota(jnp.int32, LANES)
    zeros_i = jnp.zeros((LANES,), jnp.int32)
    ones_i = jnp.ones((LANES,), jnp.int32)

    def group_step(g, n_oov):
        v = idx_v[pl.ds(g * LANES, LANES)]
        in_vocab = v < VOCAB_SIZE
        safe = jnp.where(in_vocab, v, 0)
        tile_v[pl.ds(g * LANES, LANES)] = lax.shift_right_logical(safe, 3)
        row_v[pl.ds(g * LANES, LANES)] = lax.bitwise_and(safe, 7)
        oov = v >= VOCAB_SIZE
        oov_i32 = jnp.where(in_vocab, zeros_i, ones_i)
        pos = plsc.cumsum(oov_i32)
        tgt = jnp.full((LANES,), n_oov, jnp.int32) + pos - ones_i
        rowids = lane_iota + jnp.full((LANES,), g * LANES, jnp.int32)
        plsc.store_scatter(oov_v, [tgt], rowids, mask=oov)
        return n_oov + jnp.sum(oov_i32)

    n_oov = lax.fori_loop(0, n_groups, group_step, jnp.int32(0))

    def fetch(g, carry):
        t16 = tile_v[pl.ds(g * LANES, LANES)]
        r16 = row_v[pl.ds(g * LANES, LANES)]
        for k in range(LANES):
            pltpu.async_copy(table_hbm.at[t16[k], r16[k]],
                             rows_v.at[g * LANES + k], sem.at[k])
        return carry

    lax.fori_loop(0, n_groups, fetch, jnp.int32(0))

    def drain(g, carry):
        for k in range(LANES):
            pltpu.make_async_copy(table_hbm.at[0, 0], rows_v.at[0],
                                  sem.at[k]).wait()
        return carry

    lax.fori_loop(0, n_groups, drain, jnp.int32(0))

    zeros_f = jnp.zeros((LANES,), jnp.float32)

    def zero_chunk(c, carry):
        r = oov_v[pl.ds(c * LANES, LANES)]
        lane_pos = lane_iota + jnp.full((LANES,), c * LANES, jnp.int32)
        valid = lane_pos < jnp.full((LANES,), n_oov, jnp.int32)
        for d in range(EMB_DIM):
            plsc.store_scatter(rows_v, [r, jnp.full((LANES,), d, jnp.int32)],
                               zeros_f, mask=valid)
        return carry

    lax.fori_loop(0, (n_oov + LANES - 1) // LANES, zero_chunk, jnp.int32(0))

    pltpu.sync_copy(rows_v, out_hbm.at[pl.ds(base, b_per_w)])


def kernel(words, table):
    batch = words.shape[0]
    b_per_w = batch // NUM_WORKERS
    vocab = table.shape[0]
    table3 = table.reshape(vocab // SUBLANES, SUBLANES, EMB_DIM)
    mesh = plsc.VectorSubcoreMesh(core_axis_name="c", subcore_axis_name="s")
    run = pl.kernel(
        _sc_embed_body,
        out_type=jax.ShapeDtypeStruct((batch, EMB_DIM), jnp.float32),
        mesh=mesh,
        scratch_types=[
            pltpu.VMEM((b_per_w,), jnp.int32),
            pltpu.VMEM((b_per_w,), jnp.int32),
            pltpu.VMEM((b_per_w,), jnp.int32),
            pltpu.VMEM((b_per_w + LANES,), jnp.int32),
            pltpu.VMEM((b_per_w, EMB_DIM), jnp.float32),
            pltpu.SemaphoreType.DMA((LANES,)),
        ],
        compiler_params=pltpu.CompilerParams(needs_layout_passes=False),
    )
    return run(words, table3)
